# Initial kernel scaffold; baseline (speedup 1.0000x reference)
import jax
import jax.numpy as jnp
from jax.experimental import pallas as pl


def kernel(x, edge_index, W0, W1):
    raise NotImplementedError("write your pallas kernel here")


# speedup vs baseline: 1.0146x; 1.0146x over previous
import functools

import jax
import jax.numpy as jnp
from jax.experimental import pallas as pl

_N = 10000
_D = 128
_NPAD = 10240
_BM = 512
_BN = 1024
_EDGE_MUL = 4


def _sim_block_kernel(a_ref, bt_ref, o_ref, mx_ref):
    i = pl.program_id(0)
    j = pl.program_id(1)
    s = jnp.dot(a_ref[...], bt_ref[...], preferred_element_type=jnp.float32)
    rows = i * _BM + jax.lax.broadcasted_iota(jnp.int32, (_BM, _BN), 0)
    cols = j * _BN + jax.lax.broadcasted_iota(jnp.int32, (_BM, _BN), 1)
    valid = (rows < _N) & (cols < _N)
    s = jnp.where(valid, s, -jnp.inf)
    o_ref[...] = s
    bmax = jnp.max(s, keepdims=True).reshape(1, 1)

    @pl.when((i == 0) & (j == 0))
    def _init():
        mx_ref[...] = bmax

    @pl.when((i > 0) | (j > 0))
    def _acc():
        mx_ref[...] = jnp.maximum(mx_ref[...], bmax)


def _similarity(xn):
    xnt = xn.T
    grid = (_NPAD // _BM, _NPAD // _BN)
    s, mx = pl.pallas_call(
        _sim_block_kernel,
        grid=grid,
        in_specs=[
            pl.BlockSpec((_BM, _D), lambda i, j: (i, 0)),
            pl.BlockSpec((_D, _BN), lambda i, j: (0, j)),
        ],
        out_specs=[
            pl.BlockSpec((_BM, _BN), lambda i, j: (i, j)),
            pl.BlockSpec((1, 1), lambda i, j: (0, 0)),
        ],
        out_shape=[
            jax.ShapeDtypeStruct((_NPAD, _NPAD), jnp.float32),
            jax.ShapeDtypeStruct((1, 1), jnp.float32),
        ],
    )(xn, xnt)
    return s, mx[0, 0]


def _encode(x, edge_index, Ws):
    src = edge_index[0]
    dst = edge_index[1]
    deg = jax.ops.segment_sum(
        jnp.ones((src.shape[0],), dtype=x.dtype), dst, num_segments=_N)
    inv_deg = 1.0 / jnp.clip(deg, 1.0, None)
    h = x
    for W in Ws:
        msgs = h[src]
        agg = jax.ops.segment_sum(msgs, dst, num_segments=_N) * inv_deg[:, None]
        h = jax.nn.relu(agg @ W) + h
    return h


def kernel(x, edge_index, W0, W1):
    view1_x = x
    view1_edge_index = edge_index

    xp = _encode(x, edge_index, [W0, W1])
    nrm = jnp.linalg.norm(xp, axis=1, keepdims=True)
    xn = xp / jnp.clip(nrm, 1e-12, None)
    xn_pad = jnp.zeros((_NPAD, _D), jnp.float32).at[:_N].set(xn)

    s_pad, m = _similarity(xn_pad)
    adj = s_pad[:_N, :_N]

    src = edge_index[0]
    dst = edge_index[1]
    adj = adj.at[src, dst].add(m)

    k = _EDGE_MUL * edge_index.shape[1]
    vals, flat_idx = jax.lax.top_k(adj.reshape(-1), k)
    rows = flat_idx // _N
    cols = flat_idx % _N

    seg_max = jax.ops.segment_max(vals, rows, num_segments=_N)
    e = jnp.exp(vals - seg_max[rows])
    seg_sum = jax.ops.segment_sum(e, rows, num_segments=_N)
    edge_weight = e / seg_sum[rows]

    edge_index_ = jnp.stack(
        [rows.astype(jnp.int64), cols.astype(jnp.int64)], axis=0)
    return (view1_x, view1_edge_index, x, edge_index_, edge_weight)
